# Initial kernel scaffold; baseline (speedup 1.0000x reference)
import jax
import jax.numpy as jnp
from jax.experimental import pallas as pl


def kernel(input_tokens, sampled_tokens, input_positions, seq_lens, slot_mapping, block_table, spec_tokens, accepted_num, num_seqs, num_queries, block_size):
    raise NotImplementedError("write your pallas kernel here")


# speedup vs baseline: 2.3255x; 2.3255x over previous
import functools

import jax
import jax.numpy as jnp
from jax import lax
from jax.experimental import pallas as pl
from jax.experimental.pallas import tpu as pltpu
from jax.experimental.pallas import tpu_sc as plsc

_R = 1024
_SPEC = 4
_T = 1 + _SPEC
_MAXB = 256
_BS = 128
_NW = 32
_RPW = _R // _NW
_OPW = _RPW * _T
_LANES = 16


def _body(pos_hbm, acc_hbm, samp_hbm, spec_hbm, bt_hbm,
          tok_out, pos_out, len_out, slot_out,
          pos_v, acc_v, samp_v, spec_v, bt_v,
          tok_v, posb_v, lenb_v, slotb_v):
    c = lax.axis_index("c")
    s = lax.axis_index("s")
    w = s * 2 + c
    rb = w * _RPW
    ob = w * _OPW

    pltpu.sync_copy(pos_hbm.at[pl.ds(rb, _RPW)], pos_v)
    pltpu.sync_copy(acc_hbm.at[pl.ds(rb, _RPW)], acc_v)
    pltpu.sync_copy(samp_hbm.at[pl.ds(rb * _T, _RPW * _T)], samp_v)
    pltpu.sync_copy(spec_hbm.at[pl.ds(rb * _SPEC, _RPW * _SPEC)], spec_v)
    pltpu.sync_copy(bt_hbm.at[pl.ds(rb * _MAXB, _RPW * _MAXB)], bt_v)

    for r in range(_RPW // _LANES):
        lrow = lax.iota(jnp.int32, _LANES) + r * _LANES
        pos16 = pos_v[pl.ds(r * _LANES, _LANES)]
        acc16 = acc_v[pl.ds(r * _LANES, _LANES)]
        base = pos16 + acc16
        last = plsc.load_gather(samp_v, [lrow * _T + acc16 - 1])
        for j in range(_T):
            oidx = lrow * _T + j
            p = base + j
            blk = plsc.load_gather(bt_v, [lrow * _MAXB + p // _BS])
            if j == 0:
                tok = last
            else:
                tok = plsc.load_gather(spec_v, [lrow * _SPEC + (j - 1)])
            plsc.store_scatter(tok_v, [oidx], tok)
            plsc.store_scatter(posb_v, [oidx], p)
            plsc.store_scatter(lenb_v, [oidx], p + 1)
            plsc.store_scatter(slotb_v, [oidx], blk * _BS + p % _BS)

    pltpu.sync_copy(tok_v, tok_out.at[pl.ds(ob, _OPW)])
    pltpu.sync_copy(posb_v, pos_out.at[pl.ds(ob, _OPW)])
    pltpu.sync_copy(lenb_v, len_out.at[pl.ds(ob, _OPW)])
    pltpu.sync_copy(slotb_v, slot_out.at[pl.ds(ob, _OPW)])


@functools.partial(jax.jit, static_argnames=())
def _advance(pos32, acc32, samp32, spec32, bt_flat):
    out_t = tuple(jax.ShapeDtypeStruct((_R * _T,), jnp.int32) for _ in range(4))
    scratch = (
        pltpu.VMEM((_RPW,), jnp.int32),
        pltpu.VMEM((_RPW,), jnp.int32),
        pltpu.VMEM((_RPW * _T,), jnp.int32),
        pltpu.VMEM((_RPW * _SPEC,), jnp.int32),
        pltpu.VMEM((_RPW * _MAXB,), jnp.int32),
        pltpu.VMEM((_OPW,), jnp.int32),
        pltpu.VMEM((_OPW,), jnp.int32),
        pltpu.VMEM((_OPW,), jnp.int32),
        pltpu.VMEM((_OPW,), jnp.int32),
    )
    fn = pl.kernel(
        _body,
        out_type=out_t,
        mesh=plsc.VectorSubcoreMesh(core_axis_name="c", subcore_axis_name="s"),
        scratch_types=scratch,
        compiler_params=pltpu.CompilerParams(needs_layout_passes=False),
    )
    return fn(pos32, acc32, samp32, spec32, bt_flat)


def kernel(input_tokens, sampled_tokens, input_positions, seq_lens, slot_mapping,
           block_table, spec_tokens, accepted_num, num_seqs, num_queries, block_size):
    pos32 = input_positions.astype(jnp.int32)
    acc32 = accepted_num.astype(jnp.int32)
    samp32 = sampled_tokens.astype(jnp.int32).reshape(-1)
    spec32 = spec_tokens.astype(jnp.int32).reshape(-1)
    bt_flat = block_table.astype(jnp.int32).reshape(-1)
    tok, posi, slen, slot = _advance(pos32, acc32, samp32, spec32, bt_flat)
    i64 = jnp.int64
    return tok.astype(i64), posi.astype(i64), slen.astype(i64), slot.astype(i64)


# speedup vs baseline: 2.4625x; 1.0589x over previous
import functools

import jax
import jax.numpy as jnp
from jax import lax
from jax.experimental import pallas as pl
from jax.experimental.pallas import tpu as pltpu
from jax.experimental.pallas import tpu_sc as plsc

_R = 1024
_SPEC = 4
_T = 1 + _SPEC
_MAXB = 256
_BS = 128
_NW = 32
_RPW = _R // _NW
_OPW = _RPW * _T
_LANES = 16


def _body(pos_hbm, acc_hbm, samp_hbm, spec_hbm, bt_hbm,
          tok_out, pos_out, len_out, slot_out,
          pos_v, acc_v, samp_v, spec_v, bt_v,
          tok_v, posb_v, lenb_v, slotb_v, in_sem, out_sem):
    c = lax.axis_index("c")
    s = lax.axis_index("s")
    w = s * 2 + c
    rb = w * _RPW
    ob = w * _OPW

    cps = [
        pltpu.async_copy(pos_hbm.at[pl.ds(rb, _RPW)], pos_v, in_sem),
        pltpu.async_copy(acc_hbm.at[pl.ds(rb, _RPW)], acc_v, in_sem),
        pltpu.async_copy(samp_hbm.at[pl.ds(rb * _T, _RPW * _T)], samp_v, in_sem),
        pltpu.async_copy(spec_hbm.at[pl.ds(rb * _SPEC, _RPW * _SPEC)], spec_v, in_sem),
        pltpu.async_copy(bt_hbm.at[pl.ds(rb * _MAXB, _RPW * _MAXB)], bt_v, in_sem),
    ]
    for cp in cps:
        cp.wait()

    for r in range(_RPW // _LANES):
        lrow = lax.iota(jnp.int32, _LANES) + r * _LANES
        pos16 = pos_v[pl.ds(r * _LANES, _LANES)]
        acc16 = acc_v[pl.ds(r * _LANES, _LANES)]
        base = pos16 + acc16
        last = plsc.load_gather(samp_v, [lrow * _T + acc16 - 1])
        for j in range(_T):
            oidx = lrow * _T + j
            p = base + j
            blk = plsc.load_gather(bt_v, [lrow * _MAXB + p // _BS])
            if j == 0:
                tok = last
            else:
                tok = plsc.load_gather(spec_v, [lrow * _SPEC + (j - 1)])
            plsc.store_scatter(tok_v, [oidx], tok)
            plsc.store_scatter(posb_v, [oidx], p)
            plsc.store_scatter(lenb_v, [oidx], p + 1)
            plsc.store_scatter(slotb_v, [oidx], blk * _BS + p % _BS)

    ocps = [
        pltpu.async_copy(tok_v, tok_out.at[pl.ds(ob, _OPW)], out_sem),
        pltpu.async_copy(posb_v, pos_out.at[pl.ds(ob, _OPW)], out_sem),
        pltpu.async_copy(lenb_v, len_out.at[pl.ds(ob, _OPW)], out_sem),
        pltpu.async_copy(slotb_v, slot_out.at[pl.ds(ob, _OPW)], out_sem),
    ]
    for cp in ocps:
        cp.wait()


@functools.partial(jax.jit, static_argnames=())
def _advance(pos32, acc32, samp32, spec32, bt_flat):
    out_t = tuple(jax.ShapeDtypeStruct((_R * _T,), jnp.int32) for _ in range(4))
    scratch = (
        pltpu.VMEM((_RPW,), jnp.int32),
        pltpu.VMEM((_RPW,), jnp.int32),
        pltpu.VMEM((_RPW * _T,), jnp.int32),
        pltpu.VMEM((_RPW * _SPEC,), jnp.int32),
        pltpu.VMEM((_RPW * _MAXB,), jnp.int32),
        pltpu.VMEM((_OPW,), jnp.int32),
        pltpu.VMEM((_OPW,), jnp.int32),
        pltpu.VMEM((_OPW,), jnp.int32),
        pltpu.VMEM((_OPW,), jnp.int32),
        pltpu.SemaphoreType.DMA,
        pltpu.SemaphoreType.DMA,
    )
    fn = pl.kernel(
        _body,
        out_type=out_t,
        mesh=plsc.VectorSubcoreMesh(core_axis_name="c", subcore_axis_name="s"),
        scratch_types=scratch,
        compiler_params=pltpu.CompilerParams(needs_layout_passes=False),
    )
    return fn(pos32, acc32, samp32, spec32, bt_flat)


def kernel(input_tokens, sampled_tokens, input_positions, seq_lens, slot_mapping,
           block_table, spec_tokens, accepted_num, num_seqs, num_queries, block_size):
    pos32 = input_positions.astype(jnp.int32)
    acc32 = accepted_num.astype(jnp.int32)
    samp32 = sampled_tokens.astype(jnp.int32).reshape(-1)
    spec32 = spec_tokens.astype(jnp.int32).reshape(-1)
    bt_flat = block_table.astype(jnp.int32).reshape(-1)
    tok, posi, slen, slot = _advance(pos32, acc32, samp32, spec32, bt_flat)
    i64 = jnp.int64
    return tok.astype(i64), posi.astype(i64), slen.astype(i64), slot.astype(i64)


# speedup vs baseline: 2.5832x; 1.0490x over previous
import functools

import jax
import jax.numpy as jnp
from jax import lax
from jax.experimental import pallas as pl
from jax.experimental.pallas import tpu as pltpu
from jax.experimental.pallas import tpu_sc as plsc

_R = 1024
_SPEC = 4
_T = 1 + _SPEC
_MAXB = 256
_BS = 128
_NW = 32
_RPW = _R // _NW
_OPW = _RPW * _T
_LANES = 16
_N = _R * _T

_SAMP_OFF = 0
_SPEC_OFF = _SAMP_OFF + _R * _T
_POS_OFF = _SPEC_OFF + _R * _SPEC
_ACC_OFF = _POS_OFF + _R
_IN_LEN = _ACC_OFF + _R


def _body(misc_hbm, bt_hbm, out_hbm,
          pos_v, acc_v, samp_v, spec_v, bt_v,
          tok_v, posb_v, lenb_v, slotb_v, in_sem, out_sem):
    c = lax.axis_index("c")
    s = lax.axis_index("s")
    w = s * 2 + c
    rb = w * _RPW
    ob = w * _OPW

    cps = [
        pltpu.async_copy(misc_hbm.at[pl.ds(_POS_OFF + rb, _RPW)], pos_v, in_sem),
        pltpu.async_copy(misc_hbm.at[pl.ds(_ACC_OFF + rb, _RPW)], acc_v, in_sem),
        pltpu.async_copy(misc_hbm.at[pl.ds(_SAMP_OFF + rb * _T, _RPW * _T)], samp_v, in_sem),
        pltpu.async_copy(misc_hbm.at[pl.ds(_SPEC_OFF + rb * _SPEC, _RPW * _SPEC)], spec_v, in_sem),
        pltpu.async_copy(bt_hbm.at[pl.ds(rb * _MAXB, _RPW * _MAXB)], bt_v, in_sem),
    ]
    for cp in cps:
        cp.wait()

    for r in range(_RPW // _LANES):
        lrow = lax.iota(jnp.int32, _LANES) + r * _LANES
        pos16 = pos_v[pl.ds(r * _LANES, _LANES)]
        acc16 = acc_v[pl.ds(r * _LANES, _LANES)]
        base = pos16 + acc16
        last = plsc.load_gather(samp_v, [lrow * _T + acc16 - 1])
        for j in range(_T):
            oidx = lrow * _T + j
            p = base + j
            blk = plsc.load_gather(bt_v, [lrow * _MAXB + p // _BS])
            if j == 0:
                tok = last
            else:
                tok = plsc.load_gather(spec_v, [lrow * _SPEC + (j - 1)])
            plsc.store_scatter(tok_v, [oidx], tok)
            plsc.store_scatter(posb_v, [oidx], p)
            plsc.store_scatter(lenb_v, [oidx], p + 1)
            plsc.store_scatter(slotb_v, [oidx], blk * _BS + p % _BS)

    ocps = [
        pltpu.async_copy(tok_v, out_hbm.at[pl.ds(0 * _N + ob, _OPW)], out_sem),
        pltpu.async_copy(posb_v, out_hbm.at[pl.ds(1 * _N + ob, _OPW)], out_sem),
        pltpu.async_copy(lenb_v, out_hbm.at[pl.ds(2 * _N + ob, _OPW)], out_sem),
        pltpu.async_copy(slotb_v, out_hbm.at[pl.ds(3 * _N + ob, _OPW)], out_sem),
    ]
    for cp in ocps:
        cp.wait()


@jax.jit
def _advance(misc32, bt_flat):
    scratch = (
        pltpu.VMEM((_RPW,), jnp.int32),
        pltpu.VMEM((_RPW,), jnp.int32),
        pltpu.VMEM((_RPW * _T,), jnp.int32),
        pltpu.VMEM((_RPW * _SPEC,), jnp.int32),
        pltpu.VMEM((_RPW * _MAXB,), jnp.int32),
        pltpu.VMEM((_OPW,), jnp.int32),
        pltpu.VMEM((_OPW,), jnp.int32),
        pltpu.VMEM((_OPW,), jnp.int32),
        pltpu.VMEM((_OPW,), jnp.int32),
        pltpu.SemaphoreType.DMA,
        pltpu.SemaphoreType.DMA,
    )
    fn = pl.kernel(
        _body,
        out_type=jax.ShapeDtypeStruct((4 * _N,), jnp.int32),
        mesh=plsc.VectorSubcoreMesh(core_axis_name="c", subcore_axis_name="s"),
        scratch_types=scratch,
        compiler_params=pltpu.CompilerParams(needs_layout_passes=False),
    )
    return fn(misc32, bt_flat)


def kernel(input_tokens, sampled_tokens, input_positions, seq_lens, slot_mapping,
           block_table, spec_tokens, accepted_num, num_seqs, num_queries, block_size):
    misc32 = jnp.concatenate([
        sampled_tokens.reshape(-1).astype(jnp.int32),
        spec_tokens.reshape(-1).astype(jnp.int32),
        input_positions.astype(jnp.int32),
        accepted_num.astype(jnp.int32),
    ])
    bt_flat = block_table.astype(jnp.int32).reshape(-1)
    out = _advance(misc32, bt_flat).astype(jnp.int64)
    return out[0 * _N:1 * _N], out[1 * _N:2 * _N], out[2 * _N:3 * _N], out[3 * _N:4 * _N]


# speedup vs baseline: 2.7396x; 1.0606x over previous
import functools

import jax
import jax.numpy as jnp
from jax import lax
from jax.experimental import pallas as pl
from jax.experimental.pallas import tpu as pltpu
from jax.experimental.pallas import tpu_sc as plsc

_R = 1024
_SPEC = 4
_T = 1 + _SPEC
_MAXB = 256
_BS = 128
_NW = 32
_RPW = _R // _NW
_OPW = _RPW * _T
_LANES = 16
_N = _R * _T

_SAMP_OFF = 0
_SPEC_OFF = _SAMP_OFF + _R * _T
_POS_OFF = _SPEC_OFF + _R * _SPEC
_ACC_OFF = _POS_OFF + _R
_IN_LEN = _ACC_OFF + _R


def _body(misc_hbm, bt_hbm, out_hbm,
          pos_v, acc_v, samp_v, spec_v, bt_v,
          tok_v, posb_v, lenb_v, slotb_v, in_sem, out_sem):
    c = lax.axis_index("c")
    s = lax.axis_index("s")
    w = s * 2 + c
    rb = w * _RPW
    ob = w * _OPW

    cps = [
        pltpu.async_copy(misc_hbm.at[pl.ds(_POS_OFF + rb, _RPW)], pos_v, in_sem),
        pltpu.async_copy(misc_hbm.at[pl.ds(_ACC_OFF + rb, _RPW)], acc_v, in_sem),
        pltpu.async_copy(misc_hbm.at[pl.ds(_SAMP_OFF + rb * _T, _RPW * _T)], samp_v, in_sem),
        pltpu.async_copy(misc_hbm.at[pl.ds(_SPEC_OFF + rb * _SPEC, _RPW * _SPEC)], spec_v, in_sem),
        pltpu.async_copy(bt_hbm.at[pl.ds(rb, _RPW), :], bt_v, in_sem),
    ]
    for cp in cps:
        cp.wait()

    for r in range(_RPW // _LANES):
        lrow = lax.iota(jnp.int32, _LANES) + r * _LANES
        pos16 = pos_v[pl.ds(r * _LANES, _LANES)]
        acc16 = acc_v[pl.ds(r * _LANES, _LANES)]
        base = pos16 + acc16
        last = plsc.load_gather(samp_v, [lrow * _T + acc16 - 1])
        for j in range(_T):
            oidx = lrow * _T + j
            p = base + j
            blk = plsc.load_gather(bt_v, [lrow, p // _BS])
            if j == 0:
                tok = last
            else:
                tok = plsc.load_gather(spec_v, [lrow * _SPEC + (j - 1)])
            plsc.store_scatter(tok_v, [oidx], tok)
            plsc.store_scatter(posb_v, [oidx], p)
            plsc.store_scatter(lenb_v, [oidx], p + 1)
            plsc.store_scatter(slotb_v, [oidx], blk * _BS + p % _BS)

    ocps = [
        pltpu.async_copy(tok_v, out_hbm.at[pl.ds(0 * _N + ob, _OPW)], out_sem),
        pltpu.async_copy(posb_v, out_hbm.at[pl.ds(1 * _N + ob, _OPW)], out_sem),
        pltpu.async_copy(lenb_v, out_hbm.at[pl.ds(2 * _N + ob, _OPW)], out_sem),
        pltpu.async_copy(slotb_v, out_hbm.at[pl.ds(3 * _N + ob, _OPW)], out_sem),
    ]
    for cp in ocps:
        cp.wait()


@jax.jit
def _advance(misc32, bt_flat):
    scratch = (
        pltpu.VMEM((_RPW,), jnp.int32),
        pltpu.VMEM((_RPW,), jnp.int32),
        pltpu.VMEM((_RPW * _T,), jnp.int32),
        pltpu.VMEM((_RPW * _SPEC,), jnp.int32),
        pltpu.VMEM((_RPW, _MAXB), jnp.int32),
        pltpu.VMEM((_OPW,), jnp.int32),
        pltpu.VMEM((_OPW,), jnp.int32),
        pltpu.VMEM((_OPW,), jnp.int32),
        pltpu.VMEM((_OPW,), jnp.int32),
        pltpu.SemaphoreType.DMA,
        pltpu.SemaphoreType.DMA,
    )
    fn = pl.kernel(
        _body,
        out_type=jax.ShapeDtypeStruct((4 * _N,), jnp.int32),
        mesh=plsc.VectorSubcoreMesh(core_axis_name="c", subcore_axis_name="s"),
        scratch_types=scratch,
        compiler_params=pltpu.CompilerParams(needs_layout_passes=False),
    )
    return fn(misc32, bt_flat)


def kernel(input_tokens, sampled_tokens, input_positions, seq_lens, slot_mapping,
           block_table, spec_tokens, accepted_num, num_seqs, num_queries, block_size):
    misc32 = jnp.concatenate([
        sampled_tokens.reshape(-1).astype(jnp.int32),
        spec_tokens.reshape(-1).astype(jnp.int32),
        input_positions.astype(jnp.int32),
        accepted_num.astype(jnp.int32),
    ])
    out = _advance(misc32, block_table).astype(jnp.int64)
    return out[0 * _N:1 * _N], out[1 * _N:2 * _N], out[2 * _N:3 * _N], out[3 * _N:4 * _N]
